# baseline (speedup 1.0000x reference)
import functools

import jax
import jax.numpy as jnp
from jax import lax
from jax.experimental import pallas as pl
from jax.experimental.pallas import tpu as pltpu
from jax.experimental.pallas import tpu_sc as plsc

_VOCAB = 1000000
_EMBED = 64
_WIDE = 128
_BATCH = 16384
_NEG = 20

_NC = 2
_NS = 16
_NW = _NC * _NS
_BPW = _BATCH // _NW
_CHUNK = 32
_NCHUNK = _BPW // _CHUNK
_NROWS = _CHUNK * _NEG
_NSUB = _NROWS // 128


def _sc_body(witok, ctok, ntok, w_in, w_out, pos_out, neg_out,
             widx_v, cidx_v, nidx_v, wrows, crows, nrows, pos_v, nout_v,
             tbuf, sem):
    wid = lax.axis_index("s") * _NC + lax.axis_index("c")
    base = wid * _BPW

    pltpu.sync_copy(witok.at[pl.ds(base, _BPW)], widx_v)
    pltpu.sync_copy(ctok.at[pl.ds(base, _BPW)], cidx_v)
    pltpu.sync_copy(ntok.at[pl.ds(base * _NEG, _BPW * _NEG)], nidx_v)

    def chunk_body(c, carry):
        cw = pltpu.make_async_copy(
            w_in.at[widx_v.at[pl.ds(c * _CHUNK, _CHUNK)]], wrows, sem)
        cw.start()
        cc = pltpu.make_async_copy(
            w_out.at[cidx_v.at[pl.ds(c * _CHUNK, _CHUNK)]], crows, sem)
        cc.start()
        subs = []
        for j in range(_NSUB):
            cn = pltpu.make_async_copy(
                w_out.at[nidx_v.at[pl.ds(c * _NROWS + j * 128, 128)]],
                nrows.at[pl.ds(j * 128, 128), :], sem)
            cn.start()
            subs.append(cn)
        cw.wait()
        cc.wait()
        for cn in subs:
            cn.wait()

        iota16 = lax.iota(jnp.int32, 16)
        tcol = iota16 * 16

        def load_row(ref, r):
            return [ref[r, pl.ds(16 * j, 16)] for j in range(4)]

        def dot_vec(a, b):
            return (a[0] * b[0] + a[1] * b[1]) + (a[2] * b[2] + a[3] * b[3])

        def reduce16(tvecs, start, ref):
            for i, t in enumerate(tvecs):
                plsc.store_scatter(tbuf, [tcol + i], t)
            acc = tbuf[pl.ds(0, 16)]
            for j in range(1, 16):
                acc = acc + tbuf[pl.ds(j * 16, 16)]
            ref[pl.ds(start, 16)] = acc

        def block_body(eb, carry2):
            e0 = eb * 4
            w = [load_row(wrows, e0 + i) for i in range(4)]
            for g in range(5):
                tvecs = []
                for i in range(16):
                    d = g * 16 + i
                    e_off = d // _NEG
                    nv = load_row(nrows, eb * 80 + d)
                    tvecs.append(dot_vec(w[e_off], nv))
                reduce16(tvecs, (c * _CHUNK + e0) * _NEG + g * 16, nout_v)
            return carry2

        lax.fori_loop(0, _CHUNK // 4, block_body, 0, unroll=False)

        for g in range(2):
            tvecs = []
            for i in range(16):
                e = g * 16 + i
                tvecs.append(dot_vec(load_row(wrows, e), load_row(crows, e)))
            reduce16(tvecs, c * _CHUNK + g * 16, pos_v)
        return carry

    lax.fori_loop(0, _NCHUNK, chunk_body, 0, unroll=False)

    pltpu.sync_copy(pos_v, pos_out.at[pl.ds(base, _BPW)])
    pltpu.sync_copy(nout_v, neg_out.at[pl.ds(base * _NEG, _BPW * _NEG)])


def _sc_call(witok, ctok, ntok, w_in, w_out):
    mesh = plsc.VectorSubcoreMesh(
        core_axis_name="c", subcore_axis_name="s",
        num_cores=_NC, num_subcores=_NS)
    return pl.kernel(
        _sc_body,
        out_type=(
            jax.ShapeDtypeStruct((_BATCH,), jnp.float32),
            jax.ShapeDtypeStruct((_BATCH * _NEG,), jnp.float32),
        ),
        mesh=mesh,
        compiler_params=pltpu.CompilerParams(
            needs_layout_passes=False, use_tc_tiling_on_sc=False),
        scratch_types=[
            pltpu.VMEM((_BPW,), jnp.int32),
            pltpu.VMEM((_BPW,), jnp.int32),
            pltpu.VMEM((_BPW * _NEG,), jnp.int32),
            pltpu.VMEM((_CHUNK, _EMBED), jnp.float32),
            pltpu.VMEM((_CHUNK, _WIDE), jnp.float32),
            pltpu.VMEM((_NROWS, _WIDE), jnp.float32),
            pltpu.VMEM((_BPW,), jnp.float32),
            pltpu.VMEM((_BPW * _NEG,), jnp.float32),
            pltpu.VMEM((256,), jnp.float32),
            pltpu.SemaphoreType.DMA,
        ],
    )(witok, ctok, ntok, w_in, w_out)


_TBLK = 4096


def _tr_body(in_ref, out_ref):
    eye = jnp.eye(_EMBED, dtype=jnp.float32)
    t = lax.dot_general(
        in_ref[...], eye, (((0,), (0,)), ((), ())),
        preferred_element_type=jnp.float32)
    out_ref[:, 0:_EMBED] = t


def _relayout(wt):
    return pl.pallas_call(
        _tr_body,
        grid=(pl.cdiv(_VOCAB, _TBLK),),
        in_specs=[pl.BlockSpec((_EMBED, _TBLK), lambda i: (0, i))],
        out_specs=pl.BlockSpec((_TBLK, _WIDE), lambda i: (i, 0)),
        out_shape=jax.ShapeDtypeStruct((_VOCAB, _WIDE), jnp.float32),
    )(wt)


def kernel(input_tokens, ctx_tokens, neg_tokens, W_in, W_out):
    pos, neg = _sc_call(
        input_tokens.reshape(_BATCH),
        ctx_tokens.reshape(_BATCH),
        neg_tokens.reshape(_BATCH * _NEG),
        W_in, _relayout(W_out.T))
    return pos.reshape(_BATCH, 1, 1), neg.reshape(_BATCH, 1, _NEG)


# speedup vs baseline: 1.4349x; 1.4349x over previous
import functools

import jax
import jax.numpy as jnp
from jax import lax
from jax.experimental import pallas as pl
from jax.experimental.pallas import tpu as pltpu
from jax.experimental.pallas import tpu_sc as plsc

_VOCAB = 1000000
_EMBED = 64
_WIDE = 128
_BATCH = 16384
_NEG = 20

_NC = 2
_NS = 16
_NW = _NC * _NS
_BPW = _BATCH // _NW
_CHUNK = 32
_NCHUNK = _BPW // _CHUNK
_NROWS = _CHUNK * _NEG
_NSUB = _NROWS // 128


def _sc_body(witok, ctok, ntok, w_in, w_out, pos_out, neg_out,
             widx_v, cidx_v, nidx_v, wrows, crows, nrows, pos_v, nout_v,
             tbuf, sem):
    wid = lax.axis_index("s") * _NC + lax.axis_index("c")
    base = wid * _BPW

    pltpu.sync_copy(witok.at[pl.ds(base, _BPW)], widx_v)
    pltpu.sync_copy(ctok.at[pl.ds(base, _BPW)], cidx_v)
    pltpu.sync_copy(ntok.at[pl.ds(base * _NEG, _BPW * _NEG)], nidx_v)

    def chunk_body(c, carry):
        cw = pltpu.make_async_copy(
            w_in.at[widx_v.at[pl.ds(c * _CHUNK, _CHUNK)]], wrows, sem)
        cw.start()
        cc = pltpu.make_async_copy(
            w_out.at[cidx_v.at[pl.ds(c * _CHUNK, _CHUNK)]], crows, sem)
        cc.start()
        subs = []
        for j in range(_NSUB):
            cn = pltpu.make_async_copy(
                w_out.at[nidx_v.at[pl.ds(c * _NROWS + j * 128, 128)]],
                nrows.at[pl.ds(j * 128, 128), :], sem)
            cn.start()
            subs.append(cn)
        cw.wait()
        cc.wait()
        for cn in subs:
            cn.wait()

        iota16 = lax.iota(jnp.int32, 16)
        tcol = iota16 * 16

        def load_row(ref, r):
            return [ref[r, pl.ds(16 * j, 16)] for j in range(4)]

        def dot_vec(a, b):
            return (a[0] * b[0] + a[1] * b[1]) + (a[2] * b[2] + a[3] * b[3])

        def reduce16(tvecs, start, ref):
            for i, t in enumerate(tvecs):
                plsc.store_scatter(tbuf, [tcol + i], t)
            acc = tbuf[pl.ds(0, 16)]
            for j in range(1, 16):
                acc = acc + tbuf[pl.ds(j * 16, 16)]
            ref[pl.ds(start, 16)] = acc

        def block_body(eb, carry2):
            e0 = eb * 4
            w = [load_row(wrows, e0 + i) for i in range(4)]
            for g in range(5):
                tvecs = []
                for i in range(16):
                    d = g * 16 + i
                    e_off = d // _NEG
                    nv = load_row(nrows, eb * 80 + d)
                    tvecs.append(dot_vec(w[e_off], nv))
                reduce16(tvecs, (c * _CHUNK + e0) * _NEG + g * 16, nout_v)
            return carry2

        lax.fori_loop(0, _CHUNK // 4, block_body, 0, unroll=False)

        for g in range(2):
            tvecs = []
            for i in range(16):
                e = g * 16 + i
                tvecs.append(dot_vec(load_row(wrows, e), load_row(crows, e)))
            reduce16(tvecs, c * _CHUNK + g * 16, pos_v)
        return carry

    lax.fori_loop(0, _NCHUNK, chunk_body, 0, unroll=False)

    pltpu.sync_copy(pos_v, pos_out.at[pl.ds(base, _BPW)])
    pltpu.sync_copy(nout_v, neg_out.at[pl.ds(base * _NEG, _BPW * _NEG)])


def _sc_call(witok, ctok, ntok, w_in, w_out):
    mesh = plsc.VectorSubcoreMesh(
        core_axis_name="c", subcore_axis_name="s",
        num_cores=_NC, num_subcores=_NS)
    return pl.kernel(
        _sc_body,
        out_type=(
            jax.ShapeDtypeStruct((_BATCH,), jnp.float32),
            jax.ShapeDtypeStruct((_BATCH * _NEG,), jnp.float32),
        ),
        mesh=mesh,
        compiler_params=pltpu.CompilerParams(
            needs_layout_passes=False, use_tc_tiling_on_sc=False),
        scratch_types=[
            pltpu.VMEM((_BPW,), jnp.int32),
            pltpu.VMEM((_BPW,), jnp.int32),
            pltpu.VMEM((_BPW * _NEG,), jnp.int32),
            pltpu.VMEM((_CHUNK, _WIDE), jnp.float32),
            pltpu.VMEM((_CHUNK, _WIDE), jnp.float32),
            pltpu.VMEM((_NROWS, _WIDE), jnp.float32),
            pltpu.VMEM((_BPW,), jnp.float32),
            pltpu.VMEM((_BPW * _NEG,), jnp.float32),
            pltpu.VMEM((256,), jnp.float32),
            pltpu.SemaphoreType.DMA,
        ],
    )(witok, ctok, ntok, w_in, w_out)


_TBLK = 8192


def _tr_body(in_ref, out_ref):
    eye = jnp.eye(_EMBED, dtype=jnp.float32)
    t = lax.dot_general(
        in_ref[...], eye, (((0,), (0,)), ((), ())),
        preferred_element_type=jnp.float32)
    out_ref[:, 0:_EMBED] = t


def _relayout(wt):
    return pl.pallas_call(
        _tr_body,
        grid=(pl.cdiv(_VOCAB, _TBLK),),
        in_specs=[pl.BlockSpec((_EMBED, _TBLK), lambda i: (0, i))],
        out_specs=pl.BlockSpec((_TBLK, _WIDE), lambda i: (i, 0)),
        out_shape=jax.ShapeDtypeStruct((_VOCAB, _WIDE), jnp.float32),
    )(wt)


def kernel(input_tokens, ctx_tokens, neg_tokens, W_in, W_out):
    pos, neg = _sc_call(
        input_tokens.reshape(_BATCH),
        ctx_tokens.reshape(_BATCH),
        neg_tokens.reshape(_BATCH * _NEG),
        _relayout(W_in.T), _relayout(W_out.T))
    return pos.reshape(_BATCH, 1, 1), neg.reshape(_BATCH, 1, _NEG)


# speedup vs baseline: 1.5424x; 1.0749x over previous
import functools

import jax
import jax.numpy as jnp
from jax import lax
from jax.experimental import pallas as pl
from jax.experimental.pallas import tpu as pltpu
from jax.experimental.pallas import tpu_sc as plsc

_VOCAB = 1000000
_EMBED = 64
_WIDE = 128
_BATCH = 16384
_NEG = 20

_NC = 2
_NS = 16
_NW = _NC * _NS
_BPW = _BATCH // _NW
_CHUNK = 32
_NCHUNK = _BPW // _CHUNK
_NROWS = _CHUNK * _NEG
_NSUB = _NROWS // 128


def _sc_body(witok, ctok, ntok, w_in, w_out, pos_out, neg_out,
             widx_v, cidx_v, nidx_v, wrows, crows, nrows, pos_v, nout_v,
             tbuf, sem):
    wid = lax.axis_index("s") * _NC + lax.axis_index("c")
    base = wid * _BPW

    pltpu.sync_copy(witok.at[pl.ds(base, _BPW)], widx_v)
    pltpu.sync_copy(ctok.at[pl.ds(base, _BPW)], cidx_v)
    pltpu.sync_copy(ntok.at[pl.ds(base * _NEG, _BPW * _NEG)], nidx_v)

    def chunk_body(c, carry):
        cw = pltpu.make_async_copy(
            w_in.at[widx_v.at[pl.ds(c * _CHUNK, _CHUNK)]], wrows, sem)
        cw.start()
        cc = pltpu.make_async_copy(
            w_out.at[cidx_v.at[pl.ds(c * _CHUNK, _CHUNK)]], crows, sem)
        cc.start()
        subs = []
        for j in range(_NSUB):
            cn = pltpu.make_async_copy(
                w_out.at[nidx_v.at[pl.ds(c * _NROWS + j * 128, 128)]],
                nrows.at[pl.ds(j * 128, 128), :], sem)
            cn.start()
            subs.append(cn)
        cw.wait()
        cc.wait()
        for cn in subs:
            cn.wait()

        iota16 = lax.iota(jnp.int32, 16)
        tcol = iota16 * 16

        def load_row(ref, r):
            return [ref[r, pl.ds(16 * j, 16)] for j in range(4)]

        def dot_vec(a, b):
            return (a[0] * b[0] + a[1] * b[1]) + (a[2] * b[2] + a[3] * b[3])

        def reduce16(tvecs, start, ref):
            for i, t in enumerate(tvecs):
                plsc.store_scatter(tbuf, [tcol + i], t)
            acc = tbuf[pl.ds(0, 16)]
            for j in range(1, 16):
                acc = acc + tbuf[pl.ds(j * 16, 16)]
            ref[pl.ds(start, 16)] = acc

        def block_body(eb, carry2):
            e0 = eb * 4
            w = [load_row(wrows, e0 + i) for i in range(4)]
            for g in range(5):
                tvecs = []
                for i in range(16):
                    d = g * 16 + i
                    e_off = d // _NEG
                    nv = load_row(nrows, eb * 80 + d)
                    tvecs.append(dot_vec(w[e_off], nv))
                reduce16(tvecs, (c * _CHUNK + e0) * _NEG + g * 16, nout_v)
            return carry2

        lax.fori_loop(0, _CHUNK // 4, block_body, 0, unroll=False)

        for g in range(2):
            tvecs = []
            for i in range(16):
                e = g * 16 + i
                tvecs.append(dot_vec(load_row(wrows, e), load_row(crows, e)))
            reduce16(tvecs, c * _CHUNK + g * 16, pos_v)
        return carry

    lax.fori_loop(0, _NCHUNK, chunk_body, 0, unroll=False)

    pltpu.sync_copy(pos_v, pos_out.at[pl.ds(base, _BPW)])
    pltpu.sync_copy(nout_v, neg_out.at[pl.ds(base * _NEG, _BPW * _NEG)])


def _sc_call(witok, ctok, ntok, w_in, w_out):
    mesh = plsc.VectorSubcoreMesh(
        core_axis_name="c", subcore_axis_name="s",
        num_cores=_NC, num_subcores=_NS)
    return pl.kernel(
        _sc_body,
        out_type=(
            jax.ShapeDtypeStruct((_BATCH,), jnp.float32),
            jax.ShapeDtypeStruct((_BATCH * _NEG,), jnp.float32),
        ),
        mesh=mesh,
        compiler_params=pltpu.CompilerParams(
            needs_layout_passes=False, use_tc_tiling_on_sc=False),
        scratch_types=[
            pltpu.VMEM((_BPW,), jnp.int32),
            pltpu.VMEM((_BPW,), jnp.int32),
            pltpu.VMEM((_BPW * _NEG,), jnp.int32),
            pltpu.VMEM((_CHUNK, _WIDE), jnp.float32),
            pltpu.VMEM((_CHUNK, _WIDE), jnp.float32),
            pltpu.VMEM((_NROWS, _WIDE), jnp.float32),
            pltpu.VMEM((_BPW,), jnp.float32),
            pltpu.VMEM((_BPW * _NEG,), jnp.float32),
            pltpu.VMEM((256,), jnp.float32),
            pltpu.SemaphoreType.DMA,
        ],
    )(witok, ctok, ntok, w_in, w_out)


_TBLK = 16384


def _tr_body(in_ref, out_ref):
    eye = jnp.eye(_EMBED, dtype=jnp.float32)
    t = lax.dot_general(
        in_ref[...], eye, (((0,), (0,)), ((), ())),
        preferred_element_type=jnp.float32)
    out_ref[:, 0:_EMBED] = t


def _relayout(wt):
    return pl.pallas_call(
        _tr_body,
        grid=(pl.cdiv(_VOCAB, _TBLK),),
        in_specs=[pl.BlockSpec((_EMBED, _TBLK), lambda i: (0, i))],
        out_specs=pl.BlockSpec((_TBLK, _WIDE), lambda i: (i, 0)),
        out_shape=jax.ShapeDtypeStruct((_VOCAB, _WIDE), jnp.float32),
    )(wt)


def kernel(input_tokens, ctx_tokens, neg_tokens, W_in, W_out):
    pos, neg = _sc_call(
        input_tokens.reshape(_BATCH),
        ctx_tokens.reshape(_BATCH),
        neg_tokens.reshape(_BATCH * _NEG),
        _relayout(W_in.T), _relayout(W_out.T))
    return pos.reshape(_BATCH, 1, 1), neg.reshape(_BATCH, 1, _NEG)


# speedup vs baseline: 1.5766x; 1.0222x over previous
import functools

import jax
import jax.numpy as jnp
from jax import lax
from jax.experimental import pallas as pl
from jax.experimental.pallas import tpu as pltpu
from jax.experimental.pallas import tpu_sc as plsc

_VOCAB = 1000000
_EMBED = 64
_WIDE = 128
_BATCH = 16384
_NEG = 20

_NC = 2
_NS = 16
_NW = _NC * _NS
_BPW = _BATCH // _NW
_CHUNK = 32
_NCHUNK = _BPW // _CHUNK
_NROWS = _CHUNK * _NEG
_NSUB = _NROWS // 128


def _sc_body(witok, ctok, ntok, w_in, w_out, pos_out, neg_out,
             widx_v, cidx_v, nidx_v, wrows, crows, nrows, pos_v, nout_v,
             tbuf, sem):
    wid = lax.axis_index("s") * _NC + lax.axis_index("c")
    base = wid * _BPW

    pltpu.sync_copy(witok.at[pl.ds(base, _BPW)], widx_v)
    pltpu.sync_copy(ctok.at[pl.ds(base, _BPW)], cidx_v)
    pltpu.sync_copy(ntok.at[pl.ds(base * _NEG, _BPW * _NEG)], nidx_v)

    def chunk_body(c, carry):
        cw = pltpu.make_async_copy(
            w_in.at[widx_v.at[pl.ds(c * _CHUNK, _CHUNK)]], wrows, sem)
        cw.start()
        cc = pltpu.make_async_copy(
            w_out.at[cidx_v.at[pl.ds(c * _CHUNK, _CHUNK)]], crows, sem)
        cc.start()
        subs = []
        for j in range(_NSUB):
            cn = pltpu.make_async_copy(
                w_out.at[nidx_v.at[pl.ds(c * _NROWS + j * 128, 128)]],
                nrows.at[pl.ds(j * 128, 128), :], sem)
            cn.start()
            subs.append(cn)
        cw.wait()
        cc.wait()
        for cn in subs:
            cn.wait()

        iota16 = lax.iota(jnp.int32, 16)
        tcol = iota16 * 16

        def load_row(ref, r):
            return [ref[r, pl.ds(16 * j, 16)] for j in range(4)]

        def dot_vec(a, b):
            return (a[0] * b[0] + a[1] * b[1]) + (a[2] * b[2] + a[3] * b[3])

        def reduce16(tvecs, start, ref):
            for i, t in enumerate(tvecs):
                plsc.store_scatter(tbuf, [tcol + i], t)
            acc = tbuf[pl.ds(0, 16)]
            for j in range(1, 16):
                acc = acc + tbuf[pl.ds(j * 16, 16)]
            ref[pl.ds(start, 16)] = acc

        def block_body(eb, carry2):
            e0 = eb * 4
            w = [load_row(wrows, e0 + i) for i in range(4)]
            for g in range(5):
                tvecs = []
                for i in range(16):
                    d = g * 16 + i
                    e_off = d // _NEG
                    nv = load_row(nrows, eb * 80 + d)
                    tvecs.append(dot_vec(w[e_off], nv))
                reduce16(tvecs, (c * _CHUNK + e0) * _NEG + g * 16, nout_v)
            return carry2

        lax.fori_loop(0, _CHUNK // 4, block_body, 0, unroll=False)

        for g in range(2):
            tvecs = []
            for i in range(16):
                e = g * 16 + i
                tvecs.append(dot_vec(load_row(wrows, e), load_row(crows, e)))
            reduce16(tvecs, c * _CHUNK + g * 16, pos_v)
        return carry

    lax.fori_loop(0, _NCHUNK, chunk_body, 0, unroll=False)

    pltpu.sync_copy(pos_v, pos_out.at[pl.ds(base, _BPW)])
    pltpu.sync_copy(nout_v, neg_out.at[pl.ds(base * _NEG, _BPW * _NEG)])


def _sc_call(witok, ctok, ntok, w_in, w_out):
    mesh = plsc.VectorSubcoreMesh(
        core_axis_name="c", subcore_axis_name="s",
        num_cores=_NC, num_subcores=_NS)
    return pl.kernel(
        _sc_body,
        out_type=(
            jax.ShapeDtypeStruct((_BATCH,), jnp.float32),
            jax.ShapeDtypeStruct((_BATCH * _NEG,), jnp.float32),
        ),
        mesh=mesh,
        compiler_params=pltpu.CompilerParams(
            needs_layout_passes=False, use_tc_tiling_on_sc=False),
        scratch_types=[
            pltpu.VMEM((_BPW,), jnp.int32),
            pltpu.VMEM((_BPW,), jnp.int32),
            pltpu.VMEM((_BPW * _NEG,), jnp.int32),
            pltpu.VMEM((_CHUNK, _WIDE), jnp.float32),
            pltpu.VMEM((_CHUNK, _WIDE), jnp.float32),
            pltpu.VMEM((_NROWS, _WIDE), jnp.float32),
            pltpu.VMEM((_BPW,), jnp.float32),
            pltpu.VMEM((_BPW * _NEG,), jnp.float32),
            pltpu.VMEM((256,), jnp.float32),
            pltpu.SemaphoreType.DMA,
        ],
    )(witok, ctok, ntok, w_in, w_out)


_TBLK = 32768


def _tr_body(in_ref, out_ref):
    eye = jnp.eye(_EMBED, dtype=jnp.float32)
    t = lax.dot_general(
        in_ref[...], eye, (((0,), (0,)), ((), ())),
        preferred_element_type=jnp.float32)
    out_ref[:, 0:_EMBED] = t


def _relayout(wt):
    return pl.pallas_call(
        _tr_body,
        grid=(pl.cdiv(_VOCAB, _TBLK),),
        in_specs=[pl.BlockSpec((_EMBED, _TBLK), lambda i: (0, i))],
        out_specs=pl.BlockSpec((_TBLK, _WIDE), lambda i: (i, 0)),
        out_shape=jax.ShapeDtypeStruct((_VOCAB, _WIDE), jnp.float32),
    )(wt)


def kernel(input_tokens, ctx_tokens, neg_tokens, W_in, W_out):
    pos, neg = _sc_call(
        input_tokens.reshape(_BATCH),
        ctx_tokens.reshape(_BATCH),
        neg_tokens.reshape(_BATCH * _NEG),
        _relayout(W_in.T), _relayout(W_out.T))
    return pos.reshape(_BATCH, 1, 1), neg.reshape(_BATCH, 1, _NEG)
